# Initial kernel scaffold; baseline (speedup 1.0000x reference)
import jax
import jax.numpy as jnp
from jax.experimental import pallas as pl


def kernel(actions, emb_weight):
    raise NotImplementedError("write your pallas kernel here")


# speedup vs baseline: 6.3491x; 6.3491x over previous
import functools

import jax
import jax.numpy as jnp
from jax import lax
from jax.experimental import pallas as pl
from jax.experimental.pallas import tpu as pltpu
from jax.experimental.pallas import tpu_sc as plsc

_D = 128
_NC = 2
_NS = 16
_NW = _NC * _NS
_LANES = 16

_B = 4096 * 200
_BPW = _B // _NW
_IW = 128
_KPC = 4
_CHUNK = _IW * _KPC
_NCHUNK = _BPW // _CHUNK

_mesh = plsc.VectorSubcoreMesh(core_axis_name="c", subcore_axis_name="s")


@functools.partial(
    pl.kernel,
    out_type=jax.ShapeDtypeStruct((_B, _D), jnp.float32),
    mesh=_mesh,
    scratch_types=[
        pltpu.VMEM((_KPC, _IW), jnp.int32),
        pltpu.VMEM((_CHUNK, _D), jnp.float32),
        pltpu.SemaphoreType.DMA,
    ],
)
def _emb_lookup(idx_hbm, table_hbm, out_hbm, idx_v, rows_v, sem):
    wid = lax.axis_index("s") * _NC + lax.axis_index("c")

    @pl.loop(0, _NCHUNK)
    def _chunk(g):
        irow = wid * (_BPW // _IW) + g * _KPC
        pltpu.sync_copy(idx_hbm.at[pl.ds(irow, _KPC)], idx_v)
        for k in range(_KPC):
            for i in range(_IW // _LANES):
                sl = pl.ds(i * _LANES, _LANES)
                idx_v[k, sl] = idx_v[k, sl] + 1
        for k in range(_KPC):
            pltpu.async_copy(
                table_hbm.at[idx_v.at[k]],
                rows_v.at[pl.ds(k * _IW, _IW)],
                sem,
            )
        for k in range(_KPC):
            pltpu.make_async_copy(
                table_hbm.at[idx_v.at[k]],
                rows_v.at[pl.ds(k * _IW, _IW)],
                sem,
            ).wait()
        base = wid * _BPW + g * _CHUNK
        pltpu.sync_copy(rows_v, out_hbm.at[pl.ds(base, _CHUNK)])


def kernel(actions, emb_weight):
    idx = actions.astype(jnp.int32).reshape(_B // _IW, _IW)
    out = _emb_lookup(idx, emb_weight)
    return out.reshape(actions.shape[0], actions.shape[1], _D)


# speedup vs baseline: 6.4453x; 1.0151x over previous
import functools

import jax
import jax.numpy as jnp
from jax import lax
from jax.experimental import pallas as pl
from jax.experimental.pallas import tpu as pltpu
from jax.experimental.pallas import tpu_sc as plsc

_D = 128
_NC = 2
_NS = 16
_NW = _NC * _NS
_LANES = 16

_B = 4096 * 200
_BPW = _B // _NW
_IW = 128
_KPC = 2
_CHUNK = _IW * _KPC
_NCHUNK = _BPW // _CHUNK
_NBUF = 2

_mesh = plsc.VectorSubcoreMesh(core_axis_name="c", subcore_axis_name="s")


@functools.partial(
    pl.kernel,
    out_type=jax.ShapeDtypeStruct((_B, _D), jnp.float32),
    mesh=_mesh,
    scratch_types=[
        pltpu.VMEM((_NBUF, _KPC, _IW), jnp.int32),
        pltpu.VMEM((_NBUF, _CHUNK, _D), jnp.float32),
        pltpu.SemaphoreType.DMA,
        pltpu.SemaphoreType.DMA,
        pltpu.SemaphoreType.DMA,
        pltpu.SemaphoreType.DMA,
    ],
)
def _emb_lookup(idx_hbm, table_hbm, out_hbm, idx_v, rows_v, g0, g1, o0, o1):
    wid = lax.axis_index("s") * _NC + lax.axis_index("c")
    gsem = (g0, g1)
    osem = (o0, o1)

    def load_and_fire(c, b):
        irow = wid * (_BPW // _IW) + c * _KPC
        pltpu.sync_copy(idx_hbm.at[pl.ds(irow, _KPC)], idx_v.at[b])
        for k in range(_KPC):
            for i in range(_IW // _LANES):
                sl = pl.ds(i * _LANES, _LANES)
                idx_v[b, k, sl] = idx_v[b, k, sl] + 1
        for k in range(_KPC):
            pltpu.async_copy(
                table_hbm.at[idx_v.at[b, k]],
                rows_v.at[b].at[pl.ds(k * _IW, _IW)],
                gsem[b],
            )

    def drain_and_fire_out(c, b):
        for k in range(_KPC):
            pltpu.make_async_copy(
                table_hbm.at[idx_v.at[b, k]],
                rows_v.at[b].at[pl.ds(k * _IW, _IW)],
                gsem[b],
            ).wait()
        base = wid * _BPW + c * _CHUNK
        pltpu.async_copy(rows_v.at[b], out_hbm.at[pl.ds(base, _CHUNK)], osem[b])

    def wait_out(c, b):
        base = wid * _BPW + c * _CHUNK
        pltpu.make_async_copy(
            rows_v.at[b], out_hbm.at[pl.ds(base, _CHUNK)], osem[b]
        ).wait()

    for b in range(_NBUF):
        load_and_fire(b, b)

    @pl.loop(0, (_NCHUNK - _NBUF) // _NBUF)
    def _step(t):
        g = t * _NBUF
        for b in range(_NBUF):
            drain_and_fire_out(g + b, b)
        for b in range(_NBUF):
            wait_out(g + b, b)
            load_and_fire(g + b + _NBUF, b)

    for b in range(_NBUF):
        drain_and_fire_out(_NCHUNK - _NBUF + b, b)
    for b in range(_NBUF):
        wait_out(_NCHUNK - _NBUF + b, b)


def kernel(actions, emb_weight):
    idx = actions.astype(jnp.int32).reshape(_B // _IW, _IW)
    out = _emb_lookup(idx, emb_weight)
    return out.reshape(actions.shape[0], actions.shape[1], _D)
